# Initial kernel scaffold; baseline (speedup 1.0000x reference)
import jax
import jax.numpy as jnp
from jax.experimental import pallas as pl


def kernel(x, edge_index, batch, num_graphs, W1, b1, W2, b2, W3, b3):
    raise NotImplementedError("write your pallas kernel here")


# speedup vs baseline: 4.6666x; 4.6666x over previous
import functools

import jax
import jax.numpy as jnp
from jax import lax
from jax.experimental import pallas as pl
from jax.experimental.pallas import tpu as pltpu
from jax.experimental.pallas import tpu_sc as plsc

N = 10000
NP = 10240
E = 640000
B = 128
CH = 128
NBLK = 5120
SUPB = 8
RPT = NP // 16

_MESH = dict(core_axis_name="c", subcore_axis_name="s")


def _make_agg(nslots, passes):
    @functools.partial(
        pl.kernel,
        mesh=plsc.VectorSubcoreMesh(**_MESH),
        out_type=jax.ShapeDtypeStruct((nslots, NP, CH), jnp.float32),
        scratch_types=[
            pltpu.VMEM_SHARED((NP, CH), jnp.float32),
            pltpu.VMEM((SUPB, B), jnp.int32),
            pltpu.VMEM((SUPB, B), jnp.int32),
            pltpu.VMEM((B, CH), jnp.float32),
            pltpu.SemaphoreType.DMA,
        ],
    )
    def agg(g_hbm, srcb_hbm, dstb_hbm, zeros_hbm, out_hbm,
            s_sh, srcb, dstb, rows, sem):
        cid = lax.axis_index("c")
        sid = lax.axis_index("s")
        for bpt, chunk_fn, slot_fn, ebase_fn in passes:
            c = chunk_fn(cid)
            slot = slot_fn(cid)
            ebase = ebase_fn(cid)
            pltpu.sync_copy(zeros_hbm.at[pl.ds(sid * RPT, RPT)],
                            s_sh.at[pl.ds(sid * RPT, RPT)])
            plsc.subcore_barrier()

            def body(sb, carry):
                blk0 = ebase + sid * bpt + sb * SUPB
                pltpu.sync_copy(srcb_hbm.at[pl.ds(blk0, SUPB)], srcb)
                pltpu.sync_copy(dstb_hbm.at[pl.ds(blk0, SUPB)], dstb)
                for b in range(SUPB):
                    pltpu.async_copy(g_hbm.at[c].at[srcb.at[b]], rows,
                                     sem).wait()
                    pltpu.sync_copy(rows, s_sh.at[dstb.at[b]], add=True)
                return carry

            lax.fori_loop(0, bpt // SUPB, body, 0)
            plsc.subcore_barrier()
            pltpu.sync_copy(s_sh.at[pl.ds(sid * RPT, RPT)],
                            out_hbm.at[slot].at[pl.ds(sid * RPT, RPT)])
            plsc.subcore_barrier()

    return agg


_agg_l1 = _make_agg(4, [
    (NBLK // 16, lambda cid: 2 * cid, lambda cid: 2 * cid,
     lambda cid: 0),
    (NBLK // 16, lambda cid: 2 * cid + 1, lambda cid: 2 * cid + 1,
     lambda cid: 0),
])
_agg_l23 = _make_agg(4, [
    (NBLK // 16, lambda cid: cid, lambda cid: cid, lambda cid: 0),
    (NBLK // 32, lambda cid: 2, lambda cid: 2 + cid,
     lambda cid: cid * (NBLK // 2)),
])


@functools.partial(
    pl.kernel,
    mesh=plsc.VectorSubcoreMesh(**_MESH),
    out_type=jax.ShapeDtypeStruct((2, NP, CH), jnp.float32),
    scratch_types=[
        pltpu.VMEM_SHARED((NP, CH), jnp.float32),
        pltpu.VMEM((SUPB, B), jnp.int32),
        pltpu.VMEM((B, CH), jnp.float32),
    ],
)
def _deg_kernel(dstb_hbm, ones_hbm, zeros_hbm, out_hbm, s_sh, dstb, ones_v):
    cid = lax.axis_index("c")
    sid = lax.axis_index("s")
    pltpu.sync_copy(zeros_hbm.at[pl.ds(sid * RPT, RPT)],
                    s_sh.at[pl.ds(sid * RPT, RPT)])
    pltpu.sync_copy(ones_hbm, ones_v)
    plsc.subcore_barrier()
    bpt = NBLK // 32

    def body(sb, carry):
        blk0 = cid * (NBLK // 2) + sid * bpt + sb * SUPB
        pltpu.sync_copy(dstb_hbm.at[pl.ds(blk0, SUPB)], dstb)
        for b in range(SUPB):
            pltpu.sync_copy(ones_v, s_sh.at[dstb.at[b]], add=True)
        return carry

    lax.fori_loop(0, bpt // SUPB, body, 0)
    plsc.subcore_barrier()
    pltpu.sync_copy(s_sh.at[pl.ds(sid * RPT, RPT)],
                    out_hbm.at[cid].at[pl.ds(sid * RPT, RPT)])


def _dinv_tc(deg_parts):
    BN = 512

    def body(dp_ref, out_ref):
        d = dp_ref[0, :, 0:1] + dp_ref[1, :, 0:1] + 1.0
        out_ref[...] = lax.rsqrt(d)

    return pl.pallas_call(
        body,
        grid=(NP // BN,),
        in_specs=[pl.BlockSpec((2, BN, CH), lambda i: (0, i, 0))],
        out_specs=pl.BlockSpec((BN, 1), lambda i: (i, 0)),
        out_shape=jax.ShapeDtypeStruct((NP, 1), jnp.float32),
    )(deg_parts)


def _pre1_tc(x_pad, dinv, w1):
    BN = 512

    def body(x_ref, dinv_ref, w_ref, out_ref):
        hw = jnp.dot(x_ref[...], w_ref[...],
                     preferred_element_type=jnp.float32)
        g = hw * dinv_ref[...]
        for c in range(4):
            out_ref[c] = g[:, c * CH:(c + 1) * CH]

    return pl.pallas_call(
        body,
        grid=(NP // BN,),
        in_specs=[
            pl.BlockSpec((BN, 1280), lambda i: (i, 0)),
            pl.BlockSpec((BN, 1), lambda i: (i, 0)),
            pl.BlockSpec((1280, 512), lambda i: (0, 0)),
        ],
        out_specs=pl.BlockSpec((4, BN, CH), lambda i: (0, i, 0)),
        out_shape=jax.ShapeDtypeStruct((4, NP, CH), jnp.float32),
    )(x_pad, dinv, w1)


def _combine(s_ref, g_ref, nch_in):
    parts = [s_ref[c] + g_ref[c] for c in range(nch_in - 1)]
    parts.append(s_ref[nch_in - 1] + s_ref[nch_in] + g_ref[nch_in - 1])
    return jnp.concatenate(parts, axis=1)


def _mid_tc(s_r, g_r, dinv, b_row, w_next, nch_in, nch_out, split_in):
    BN = 512
    din = nch_in * CH
    dout = nch_out * CH
    nslots = nch_in + 1 if split_in else nch_in

    def body(s_ref, g_ref, dinv_ref, b_ref, w_ref, out_ref):
        if split_in:
            agg = _combine(s_ref, g_ref, nch_in)
        else:
            agg = jnp.concatenate(
                [s_ref[c] + g_ref[c] for c in range(nch_in)], axis=1)
        h = jnp.maximum(agg * dinv_ref[...] + b_ref[...], 0.0)
        gn = jnp.dot(h, w_ref[...],
                     preferred_element_type=jnp.float32) * dinv_ref[...]
        for c in range(nch_out):
            out_ref[c] = gn[:, c * CH:(c + 1) * CH]

    return pl.pallas_call(
        body,
        grid=(NP // BN,),
        in_specs=[
            pl.BlockSpec((nslots, BN, CH), lambda i: (0, i, 0)),
            pl.BlockSpec((nch_in, BN, CH), lambda i: (0, i, 0)),
            pl.BlockSpec((BN, 1), lambda i: (i, 0)),
            pl.BlockSpec((1, din), lambda i: (0, 0)),
            pl.BlockSpec((din, dout), lambda i: (0, 0)),
        ],
        out_specs=pl.BlockSpec((nch_out, BN, CH), lambda i: (0, i, 0)),
        out_shape=jax.ShapeDtypeStruct((nch_out, NP, CH), jnp.float32),
    )(s_r, g_r, dinv, b_row, w_next)


def _post_tc(s_r, g_r, dinv, b_row, nch_in):
    BN = 512
    din = nch_in * CH

    def body(s_ref, g_ref, dinv_ref, b_ref, out_ref):
        agg = _combine(s_ref, g_ref, nch_in)
        out_ref[...] = jnp.maximum(agg * dinv_ref[...] + b_ref[...], 0.0)

    return pl.pallas_call(
        body,
        grid=(NP // BN,),
        in_specs=[
            pl.BlockSpec((nch_in + 1, BN, CH), lambda i: (0, i, 0)),
            pl.BlockSpec((nch_in, BN, CH), lambda i: (0, i, 0)),
            pl.BlockSpec((BN, 1), lambda i: (i, 0)),
            pl.BlockSpec((1, din), lambda i: (0, 0)),
        ],
        out_specs=pl.BlockSpec((BN, din), lambda i: (i, 0)),
        out_shape=jax.ShapeDtypeStruct((NP, din), jnp.float32),
    )(s_r, g_r, dinv, b_row)


def kernel(x, edge_index, batch, num_graphs, W1, b1, W2, b2, W3, b3):
    f32 = jnp.float32
    x_pad = jnp.zeros((NP, 1280), f32).at[:N].set(x)
    pad = jnp.full((NBLK * B - E,), N, jnp.int32)
    srcb = jnp.concatenate([edge_index[0], pad]).reshape(NBLK, B)
    dstb = jnp.concatenate([edge_index[1], pad]).reshape(NBLK, B)
    zeros128 = jnp.zeros((NP, CH), f32)
    ones128 = jnp.ones((B, CH), f32)
    w2p = jnp.zeros((512, 384), f32).at[:, :300].set(W2)
    w3p = jnp.zeros((384, 384), f32).at[:300, :300].set(W3)
    b1r = b1.reshape(1, 512)
    b2r = jnp.zeros((1, 384), f32).at[0, :300].set(b2)
    b3r = jnp.zeros((1, 384), f32).at[0, :300].set(b3)

    deg_parts = _deg_kernel(dstb, ones128, zeros128)
    dinv = _dinv_tc(deg_parts)

    g1 = _pre1_tc(x_pad, dinv, W1)
    s1 = _agg_l1(g1, srcb, dstb, zeros128)
    g2 = _mid_tc(s1, g1, dinv, b1r, w2p, 4, 3, False)
    s2 = _agg_l23(g2, srcb, dstb, zeros128)
    g3 = _mid_tc(s2, g2, dinv, b2r, w3p, 3, 3, True)
    s3 = _agg_l23(g3, srcb, dstb, zeros128)
    h3 = _post_tc(s3, g3, dinv, b3r, 3)
    return h3[:N, :300]


# speedup vs baseline: 5.3521x; 1.1469x over previous
import functools

import jax
import jax.numpy as jnp
from jax import lax
from jax.experimental import pallas as pl
from jax.experimental.pallas import tpu as pltpu
from jax.experimental.pallas import tpu_sc as plsc

N = 10000
NP = 10240
E = 640000
B = 128
CH = 128
NBLK = 5120
SUPB = 32
RPT = NP // 16

_MESH = dict(core_axis_name="c", subcore_axis_name="s")


def _make_agg(nslots, passes):
    @functools.partial(
        pl.kernel,
        mesh=plsc.VectorSubcoreMesh(**_MESH),
        out_type=jax.ShapeDtypeStruct((nslots, NP, CH), jnp.float32),
        scratch_types=[
            pltpu.VMEM_SHARED((NP, CH), jnp.float32),
            pltpu.VMEM((SUPB, B), jnp.int32),
            pltpu.VMEM((SUPB, B), jnp.int32),
            pltpu.VMEM((2, B, CH), jnp.float32),
            pltpu.SemaphoreType.DMA,
            pltpu.SemaphoreType.DMA,
        ],
    )
    def agg(g_hbm, srcb_hbm, dstb_hbm, zeros_hbm, out_hbm,
            s_sh, srcb, dstb, rows, sem_g, sem_s):
        cid = lax.axis_index("c")
        sid = lax.axis_index("s")
        for bpt, chunk_fn, slot_fn, ebase_fn in passes:
            c = chunk_fn(cid)
            slot = slot_fn(cid)
            ebase = ebase_fn(cid)
            g_tab = g_hbm.at[c]

            def issue_g(i, bb):
                pltpu.async_copy(g_tab.at[srcb.at[i]], rows.at[bb], sem_g)

            def issue_s(i, bb):
                pltpu.async_copy(rows.at[bb], s_sh.at[dstb.at[i]],
                                 sem_s, add=True)

            def drain(sem):
                pltpu.make_async_copy(zeros_hbm.at[pl.ds(0, B)],
                                      rows.at[0], sem).wait()

            pltpu.sync_copy(zeros_hbm.at[pl.ds(sid * RPT, RPT)],
                            s_sh.at[pl.ds(sid * RPT, RPT)])
            plsc.subcore_barrier()

            def super_body(sb, carry):
                blk0 = ebase + sid * bpt + sb * SUPB
                pltpu.sync_copy(srcb_hbm.at[pl.ds(blk0, SUPB)], srcb)
                pltpu.sync_copy(dstb_hbm.at[pl.ds(blk0, SUPB)], dstb)
                issue_g(0, 0)
                drain(sem_g)
                issue_g(1, 1)
                issue_s(0, 0)

                def gbody(g, carry2):
                    drain(sem_g)
                    drain(sem_s)
                    issue_g(g + 1, lax.rem(g + 1, 2))
                    issue_s(g, lax.rem(g, 2))
                    return carry2

                lax.fori_loop(1, SUPB - 1, gbody, 0)
                drain(sem_g)
                drain(sem_s)
                issue_s(SUPB - 1, (SUPB - 1) % 2)
                drain(sem_s)
                return carry

            lax.fori_loop(0, bpt // SUPB, super_body, 0)
            plsc.subcore_barrier()
            pltpu.sync_copy(s_sh.at[pl.ds(sid * RPT, RPT)],
                            out_hbm.at[slot].at[pl.ds(sid * RPT, RPT)])
            plsc.subcore_barrier()

    return agg


_agg_l1 = _make_agg(4, [
    (NBLK // 16, lambda cid: 2 * cid, lambda cid: 2 * cid,
     lambda cid: 0),
    (NBLK // 16, lambda cid: 2 * cid + 1, lambda cid: 2 * cid + 1,
     lambda cid: 0),
])
_agg_l23 = _make_agg(4, [
    (NBLK // 16, lambda cid: cid, lambda cid: cid, lambda cid: 0),
    (NBLK // 32, lambda cid: 2, lambda cid: 2 + cid,
     lambda cid: cid * (NBLK // 2)),
])


@functools.partial(
    pl.kernel,
    mesh=plsc.VectorSubcoreMesh(**_MESH),
    out_type=jax.ShapeDtypeStruct((2, NP, CH), jnp.float32),
    scratch_types=[
        pltpu.VMEM_SHARED((NP, CH), jnp.float32),
        pltpu.VMEM((SUPB, B), jnp.int32),
        pltpu.VMEM((B, CH), jnp.float32),
    ],
)
def _deg_kernel(dstb_hbm, ones_hbm, zeros_hbm, out_hbm, s_sh, dstb, ones_v):
    cid = lax.axis_index("c")
    sid = lax.axis_index("s")
    pltpu.sync_copy(zeros_hbm.at[pl.ds(sid * RPT, RPT)],
                    s_sh.at[pl.ds(sid * RPT, RPT)])
    pltpu.sync_copy(ones_hbm, ones_v)
    plsc.subcore_barrier()
    bpt = NBLK // 32

    def body(sb, carry):
        blk0 = cid * (NBLK // 2) + sid * bpt + sb * SUPB
        pltpu.sync_copy(dstb_hbm.at[pl.ds(blk0, SUPB)], dstb)
        for b in range(SUPB):
            pltpu.sync_copy(ones_v, s_sh.at[dstb.at[b]], add=True)
        return carry

    lax.fori_loop(0, bpt // SUPB, body, 0)
    plsc.subcore_barrier()
    pltpu.sync_copy(s_sh.at[pl.ds(sid * RPT, RPT)],
                    out_hbm.at[cid].at[pl.ds(sid * RPT, RPT)])


def _dinv_tc(deg_parts):
    BN = 512

    def body(dp_ref, out_ref):
        d = dp_ref[0, :, 0:1] + dp_ref[1, :, 0:1] + 1.0
        out_ref[...] = lax.rsqrt(d)

    return pl.pallas_call(
        body,
        grid=(NP // BN,),
        in_specs=[pl.BlockSpec((2, BN, CH), lambda i: (0, i, 0))],
        out_specs=pl.BlockSpec((BN, 1), lambda i: (i, 0)),
        out_shape=jax.ShapeDtypeStruct((NP, 1), jnp.float32),
    )(deg_parts)


def _pre1_tc(x_pad, dinv, w1):
    BN = 512

    def body(x_ref, dinv_ref, w_ref, out_ref):
        hw = jnp.dot(x_ref[...], w_ref[...],
                     preferred_element_type=jnp.float32)
        g = hw * dinv_ref[...]
        for c in range(4):
            out_ref[c] = g[:, c * CH:(c + 1) * CH]

    return pl.pallas_call(
        body,
        grid=(NP // BN,),
        in_specs=[
            pl.BlockSpec((BN, 1280), lambda i: (i, 0)),
            pl.BlockSpec((BN, 1), lambda i: (i, 0)),
            pl.BlockSpec((1280, 512), lambda i: (0, 0)),
        ],
        out_specs=pl.BlockSpec((4, BN, CH), lambda i: (0, i, 0)),
        out_shape=jax.ShapeDtypeStruct((4, NP, CH), jnp.float32),
    )(x_pad, dinv, w1)


def _combine(s_ref, g_ref, nch_in):
    parts = [s_ref[c] + g_ref[c] for c in range(nch_in - 1)]
    parts.append(s_ref[nch_in - 1] + s_ref[nch_in] + g_ref[nch_in - 1])
    return jnp.concatenate(parts, axis=1)


def _mid_tc(s_r, g_r, dinv, b_row, w_next, nch_in, nch_out, split_in):
    BN = 512
    din = nch_in * CH
    dout = nch_out * CH
    nslots = nch_in + 1 if split_in else nch_in

    def body(s_ref, g_ref, dinv_ref, b_ref, w_ref, out_ref):
        if split_in:
            agg = _combine(s_ref, g_ref, nch_in)
        else:
            agg = jnp.concatenate(
                [s_ref[c] + g_ref[c] for c in range(nch_in)], axis=1)
        h = jnp.maximum(agg * dinv_ref[...] + b_ref[...], 0.0)
        gn = jnp.dot(h, w_ref[...],
                     preferred_element_type=jnp.float32) * dinv_ref[...]
        for c in range(nch_out):
            out_ref[c] = gn[:, c * CH:(c + 1) * CH]

    return pl.pallas_call(
        body,
        grid=(NP // BN,),
        in_specs=[
            pl.BlockSpec((nslots, BN, CH), lambda i: (0, i, 0)),
            pl.BlockSpec((nch_in, BN, CH), lambda i: (0, i, 0)),
            pl.BlockSpec((BN, 1), lambda i: (i, 0)),
            pl.BlockSpec((1, din), lambda i: (0, 0)),
            pl.BlockSpec((din, dout), lambda i: (0, 0)),
        ],
        out_specs=pl.BlockSpec((nch_out, BN, CH), lambda i: (0, i, 0)),
        out_shape=jax.ShapeDtypeStruct((nch_out, NP, CH), jnp.float32),
    )(s_r, g_r, dinv, b_row, w_next)


def _post_tc(s_r, g_r, dinv, b_row, nch_in):
    BN = 512
    din = nch_in * CH

    def body(s_ref, g_ref, dinv_ref, b_ref, out_ref):
        agg = _combine(s_ref, g_ref, nch_in)
        out_ref[...] = jnp.maximum(agg * dinv_ref[...] + b_ref[...], 0.0)

    return pl.pallas_call(
        body,
        grid=(NP // BN,),
        in_specs=[
            pl.BlockSpec((nch_in + 1, BN, CH), lambda i: (0, i, 0)),
            pl.BlockSpec((nch_in, BN, CH), lambda i: (0, i, 0)),
            pl.BlockSpec((BN, 1), lambda i: (i, 0)),
            pl.BlockSpec((1, din), lambda i: (0, 0)),
        ],
        out_specs=pl.BlockSpec((BN, din), lambda i: (i, 0)),
        out_shape=jax.ShapeDtypeStruct((NP, din), jnp.float32),
    )(s_r, g_r, dinv, b_row)


def kernel(x, edge_index, batch, num_graphs, W1, b1, W2, b2, W3, b3):
    f32 = jnp.float32
    x_pad = jnp.zeros((NP, 1280), f32).at[:N].set(x)
    pad = jnp.full((NBLK * B - E,), N, jnp.int32)
    srcb = jnp.concatenate([edge_index[0], pad]).reshape(NBLK, B)
    dstb = jnp.concatenate([edge_index[1], pad]).reshape(NBLK, B)
    zeros128 = jnp.zeros((NP, CH), f32)
    ones128 = jnp.ones((B, CH), f32)
    w2p = jnp.zeros((512, 384), f32).at[:, :300].set(W2)
    w3p = jnp.zeros((384, 384), f32).at[:300, :300].set(W3)
    b1r = b1.reshape(1, 512)
    b2r = jnp.zeros((1, 384), f32).at[0, :300].set(b2)
    b3r = jnp.zeros((1, 384), f32).at[0, :300].set(b3)

    deg_parts = _deg_kernel(dstb, ones128, zeros128)
    dinv = _dinv_tc(deg_parts)

    g1 = _pre1_tc(x_pad, dinv, W1)
    s1 = _agg_l1(g1, srcb, dstb, zeros128)
    g2 = _mid_tc(s1, g1, dinv, b1r, w2p, 4, 3, False)
    s2 = _agg_l23(g2, srcb, dstb, zeros128)
    g3 = _mid_tc(s2, g2, dinv, b2r, w3p, 3, 3, True)
    s3 = _agg_l23(g3, srcb, dstb, zeros128)
    h3 = _post_tc(s3, g3, dinv, b3r, 3)
    return h3[:N, :300]
